# baseline (speedup 1.0000x reference)
import functools

import jax
import jax.numpy as jnp
from jax import lax
from jax.experimental import pallas as pl
from jax.experimental.pallas import tpu as pltpu
from jax.experimental.pallas import tpu_sc as plsc

N = 100000
E = 1600000
IN_DIM = 128
HID = 32
NCLS = 40
NEG = 0.2
EPS = 1e-9

NP = 100352
NT = 16
NC = 2
NW = NC * NT
STRIPE = NP // NT
ZR = 392
CH = 2000
CHA = 800
CHS = 400
EPW = E // NW
EPT = E // NT

_mesh = plsc.VectorSubcoreMesh(core_axis_name="c", subcore_axis_name="s")
_sc_params = pltpu.CompilerParams(
    needs_layout_passes=False, use_tc_tiling_on_sc=False)
f32 = jnp.float32
i32 = jnp.int32


@functools.partial(
    pl.kernel,
    out_type=jax.ShapeDtypeStruct((E,), f32),
    mesh=_mesh,
    compiler_params=_sc_params,
    scratch_types=[
        pltpu.VMEM((NP,), f32),
        pltpu.VMEM((CH,), i32),
        pltpu.VMEM((CH,), f32),
    ],
)
def _sc_gather_el(el_hbm, src_hbm, s_hbm, tab, idxb, outb):
    cid = lax.axis_index("c")
    sid = lax.axis_index("s")
    wid = cid * NT + sid
    pltpu.sync_copy(el_hbm, tab)
    base0 = wid * EPW

    def chunk(ci, _):
        b = base0 + ci * CH
        pltpu.sync_copy(src_hbm.at[pl.ds(b, CH)], idxb)

        def grp(g, _):
            o = g * 16
            idx = idxb[pl.ds(o, 16)]
            outb[pl.ds(o, 16)] = plsc.load_gather(tab, [idx])
            return 0

        lax.fori_loop(0, CH // 16, grp, 0, unroll=4)
        pltpu.sync_copy(outb, s_hbm.at[pl.ds(b, CH)])
        return 0

    lax.fori_loop(0, EPW // CH, chunk, 0)


@functools.partial(
    pl.kernel,
    out_type=(
        jax.ShapeDtypeStruct((E,), f32),
        jax.ShapeDtypeStruct((NC, NP), f32),
    ),
    mesh=_mesh,
    compiler_params=_sc_params,
    scratch_types=[
        pltpu.VMEM((NP,), f32),
        pltpu.VMEM((CH,), i32),
        pltpu.VMEM((CH,), f32),
        pltpu.VMEM((CH,), f32),
        pltpu.VMEM((STRIPE,), f32),
        pltpu.VMEM_SHARED((NP,), f32),
    ],
)
def _sc_score(er_hbm, s_hbm, dst_hbm, ee_hbm, den_hbm,
              tab, dstb, sb, eeb, stage, den_sp):
    cid = lax.axis_index("c")
    sid = lax.axis_index("s")
    wid = cid * NT + sid
    pltpu.sync_copy(er_hbm, tab)

    def zgrp(g, _):
        stage[pl.ds(g * 16, 16)] = jnp.zeros((16,), f32)
        return 0

    lax.fori_loop(0, STRIPE // 16, zgrp, 0, unroll=4)
    pltpu.sync_copy(stage, den_sp.at[pl.ds(sid * STRIPE, STRIPE)])
    plsc.subcore_barrier()

    base0 = wid * EPW

    def chunk(ci, _):
        b = base0 + ci * CH
        pltpu.sync_copy(dst_hbm.at[pl.ds(b, CH)], dstb)
        pltpu.sync_copy(s_hbm.at[pl.ds(b, CH)], sb)

        def grp(g, _):
            o = g * 16
            idx = dstb[pl.ds(o, 16)]
            e = sb[pl.ds(o, 16)] + plsc.load_gather(tab, [idx])
            e = jnp.where(e > 0, e, NEG * e)
            eeb[pl.ds(o, 16)] = jnp.exp(e)
            return 0

        lax.fori_loop(0, CH // 16, grp, 0, unroll=4)
        pltpu.sync_copy(eeb, ee_hbm.at[pl.ds(b, CH)])
        pltpu.sync_copy(eeb, den_sp.at[dstb], add=True)
        return 0

    lax.fori_loop(0, EPW // CH, chunk, 0)

    plsc.subcore_barrier()
    off = sid * STRIPE
    pltpu.sync_copy(den_sp.at[pl.ds(off, STRIPE)], stage)
    pltpu.sync_copy(stage, den_hbm.at[cid, pl.ds(off, STRIPE)])


@functools.partial(
    pl.kernel,
    out_type=jax.ShapeDtypeStruct((NC, NP, 16), f32),
    mesh=_mesh,
    compiler_params=_sc_params,
    scratch_types=[
        pltpu.VMEM((CHA,), i32),
        pltpu.VMEM((CHA,), i32),
        pltpu.VMEM((CHA,), f32),
        pltpu.VMEM((CHA, 16), f32),
        pltpu.VMEM((ZR, 16), f32),
        pltpu.VMEM_SHARED((NP, 16), f32),
        pltpu.SemaphoreType.DMA,
    ],
)
def _sc_agg(fab_hbm, src_hbm, dst_hbm, w_hbm, acc_hbm,
            idxs, idxd, wb, rows, stage, acc_sp, sem):
    cid = lax.axis_index("c")
    sid = lax.axis_index("s")

    def zgrp(g, _):
        stage[g, :] = jnp.zeros((16,), f32)
        return 0

    lax.fori_loop(0, ZR, zgrp, 0, unroll=4)
    for k in range(NT):
        pltpu.sync_copy(stage, acc_sp.at[pl.ds(sid * STRIPE + k * ZR, ZR)])
    plsc.subcore_barrier()

    tab = fab_hbm.at[cid]
    base0 = sid * EPT

    def chunk(ci, _):
        b = base0 + ci * CHA
        pltpu.sync_copy(src_hbm.at[pl.ds(b, CHA)], idxs)
        pltpu.sync_copy(dst_hbm.at[pl.ds(b, CHA)], idxd)
        pltpu.sync_copy(w_hbm.at[pl.ds(b, CHA)], wb)
        pltpu.async_copy(tab.at[idxs], rows, sem).wait()

        def grp(g, _):
            o = g * 16
            av = wb[pl.ds(o, 16)]
            for j in range(16):
                rows[o + j, :] = rows[o + j, :] * av[j]
            return 0

        lax.fori_loop(0, CHA // 16, grp, 0)
        pltpu.sync_copy(rows, acc_sp.at[idxd], add=True)
        return 0

    lax.fori_loop(0, EPT // CHA, chunk, 0)

    plsc.subcore_barrier()
    for k in range(NT):
        off = sid * STRIPE + k * ZR
        pltpu.sync_copy(acc_sp.at[pl.ds(off, ZR)], stage)
        pltpu.sync_copy(stage, acc_hbm.at[cid, pl.ds(off, ZR)])


@functools.partial(
    pl.kernel,
    out_type=jax.ShapeDtypeStruct((NC, NP, 16), f32),
    mesh=_mesh,
    compiler_params=_sc_params,
    scratch_types=[
        pltpu.VMEM((CHS,), i32),
        pltpu.VMEM((CHS,), i32),
        pltpu.VMEM((CHS,), f32),
        pltpu.VMEM((CHS, 16), f32),
        pltpu.VMEM((ZR, 16), f32),
        pltpu.VMEM_SHARED((NP, 16), f32),
        pltpu.SemaphoreType.DMA,
    ],
)
def _sc_agg_split(tab_hbm, src_hbm, dst_hbm, w_hbm, acc_hbm,
                  idxs, idxd, wb, rows, stage, acc_sp, sem):
    cid = lax.axis_index("c")
    sid = lax.axis_index("s")

    def zgrp(g, _):
        stage[g, :] = jnp.zeros((16,), f32)
        return 0

    lax.fori_loop(0, ZR, zgrp, 0, unroll=4)
    for k in range(NT):
        pltpu.sync_copy(stage, acc_sp.at[pl.ds(sid * STRIPE + k * ZR, ZR)])
    plsc.subcore_barrier()

    base0 = (cid * NT + sid) * EPW

    def chunk(ci, _):
        b = base0 + ci * CHS
        pltpu.sync_copy(src_hbm.at[pl.ds(b, CHS)], idxs)
        pltpu.sync_copy(dst_hbm.at[pl.ds(b, CHS)], idxd)
        pltpu.sync_copy(w_hbm.at[pl.ds(b, CHS)], wb)
        pltpu.async_copy(tab_hbm.at[idxs], rows, sem).wait()

        def grp(g, _):
            o = g * 16
            av = wb[pl.ds(o, 16)]
            for j in range(16):
                rows[o + j, :] = rows[o + j, :] * av[j]
            return 0

        lax.fori_loop(0, CHS // 16, grp, 0)
        pltpu.sync_copy(rows, acc_sp.at[idxd], add=True)
        return 0

    lax.fori_loop(0, EPW // CHS, chunk, 0)

    plsc.subcore_barrier()
    for k in range(NT):
        off = sid * STRIPE + k * ZR
        pltpu.sync_copy(acc_sp.at[pl.ds(off, ZR)], stage)
        pltpu.sync_copy(stage, acc_hbm.at[cid, pl.ds(off, ZR)])


@functools.partial(
    pl.kernel,
    out_type=jax.ShapeDtypeStruct((E,), f32),
    mesh=_mesh,
    compiler_params=_sc_params,
    scratch_types=[
        pltpu.VMEM((NP,), f32),
        pltpu.VMEM((2048,), f32),
        pltpu.VMEM((2048,), f32),
        pltpu.VMEM((CH,), i32),
        pltpu.VMEM((CH,), f32),
        pltpu.VMEM((CH,), f32),
    ],
)
def _sc_alpha(den_hbm, ee_hbm, dst_hbm, alpha_hbm, tab, da, db, dstb, eeb, ab):
    cid = lax.axis_index("c")
    sid = lax.axis_index("s")
    wid = cid * NT + sid

    def tchunk(ci, _):
        o = ci * 2048
        pltpu.sync_copy(den_hbm.at[0, pl.ds(o, 2048)], da)
        pltpu.sync_copy(den_hbm.at[1, pl.ds(o, 2048)], db)

        def tgrp(g, _):
            go = g * 16
            tab[pl.ds(o + go, 16)] = (da[pl.ds(go, 16)] + db[pl.ds(go, 16)]
                                      + EPS)
            return 0

        lax.fori_loop(0, 2048 // 16, tgrp, 0, unroll=4)
        return 0

    lax.fori_loop(0, NP // 2048, tchunk, 0)

    base0 = wid * EPW

    def chunk(ci, _):
        b = base0 + ci * CH
        pltpu.sync_copy(dst_hbm.at[pl.ds(b, CH)], dstb)
        pltpu.sync_copy(ee_hbm.at[pl.ds(b, CH)], eeb)

        def grp(g, _):
            o = g * 16
            idx = dstb[pl.ds(o, 16)]
            ab[pl.ds(o, 16)] = eeb[pl.ds(o, 16)] / plsc.load_gather(tab, [idx])
            return 0

        lax.fori_loop(0, CH // 16, grp, 0, unroll=4)
        pltpu.sync_copy(ab, alpha_hbm.at[pl.ds(b, CH)])
        return 0

    lax.fori_loop(0, EPW // CH, chunk, 0)


_BR = 512
_GRID = NP // _BR


def _tc_call(body, out_shapes, in_specs, out_specs, *args):
    return pl.pallas_call(
        body,
        grid=(_GRID,),
        in_specs=in_specs,
        out_specs=out_specs,
        out_shape=out_shapes,
    )(*args)


def _full(shape):
    return pl.BlockSpec(shape, lambda i: tuple(0 for _ in shape))


def _rows(shape):
    nd = len(shape)
    return pl.BlockSpec(shape, lambda i, nd=nd: tuple(
        i if d == nd - 2 else 0 for d in range(nd)))


def _rows_last(shape):
    nd = len(shape)
    return pl.BlockSpec(shape, lambda i, nd=nd: tuple(
        i if d == nd - 1 else 0 for d in range(nd)))


def _tc0_body(x_ref, w_ref, alt_ref, art_ref, fab_ref, el_ref, er_ref):
    feat = jnp.dot(x_ref[...], w_ref[...], preferred_element_type=f32)
    fab_ref[0] = feat[:, :16]
    fab_ref[1] = feat[:, 16:]
    el_ref[...] = jnp.dot(feat, alt_ref[...], preferred_element_type=f32)
    er_ref[...] = jnp.dot(feat, art_ref[...], preferred_element_type=f32)


def _tc_layer0(xp, W0, al0, ar0):
    return _tc_call(
        _tc0_body,
        (jax.ShapeDtypeStruct((NC, NP, 16), f32),
         jax.ShapeDtypeStruct((NP, 1), f32),
         jax.ShapeDtypeStruct((NP, 1), f32)),
        [_rows((_BR, IN_DIM)), _full((IN_DIM, HID)), _full((HID, 1)),
         _full((HID, 1))],
        (_rows((NC, _BR, 16)), _rows((_BR, 1)), _rows((_BR, 1))),
        xp, W0, al0.T, ar0.T)


def _tc1_body(acc_ref, den_ref, b_ref, w_ref, alt_ref, art_ref,
              res_ref, fab_ref, el_ref, er_ref, h_ref):
    den = den_ref[0] + den_ref[1] + EPS
    h = jnp.concatenate([acc_ref[0], acc_ref[1]], axis=1) / den[:, None]
    h = h + b_ref[...]
    if res_ref is not None:
        h = h + res_ref[...]
    h_ref[...] = h
    feat = jnp.dot(h, w_ref[...], preferred_element_type=f32)
    fab_ref[0] = feat[:, :16]
    fab_ref[1] = feat[:, 16:]
    el_ref[...] = jnp.dot(feat, alt_ref[...], preferred_element_type=f32)
    er_ref[...] = jnp.dot(feat, art_ref[...], preferred_element_type=f32)


def _tc_layer1(acc0, den0, b0, W1, al1, ar1):
    body = functools.partial(_tc1_body, res_ref=None)

    def b2(acc_ref, den_ref, b_ref, w_ref, alt_ref, art_ref,
           fab_ref, el_ref, er_ref, h_ref):
        _tc1_body(acc_ref, den_ref, b_ref, w_ref, alt_ref, art_ref,
                  None, fab_ref, el_ref, er_ref, h_ref)

    return _tc_call(
        b2,
        (jax.ShapeDtypeStruct((NC, NP, 16), f32),
         jax.ShapeDtypeStruct((NP, 1), f32),
         jax.ShapeDtypeStruct((NP, 1), f32),
         jax.ShapeDtypeStruct((NP, HID), f32)),
        [_rows((NC, _BR, 16)), _rows_last((NC, _BR)), _full((1, HID)),
         _full((HID, HID)), _full((HID, 1)), _full((HID, 1))],
        (_rows((NC, _BR, 16)), _rows((_BR, 1)), _rows((_BR, 1)),
         _rows((_BR, HID))),
        acc0, den0, b0.reshape(1, HID), W1, al1.T, ar1.T)


def _tc2_body(acc_ref, den_ref, h0_ref, b_ref, w_ref, rw_ref, alt_ref,
              art_ref, fab_ref, fc_ref, el_ref, er_ref, res_ref, pri_ref):
    den = den_ref[0] + den_ref[1] + EPS
    h = jnp.concatenate([acc_ref[0], acc_ref[1]], axis=1) / den[:, None]
    h = h + h0_ref[...] + b_ref[...]
    pri_ref[...] = jnp.mean(h, axis=1, keepdims=True)
    feat = jnp.dot(h, w_ref[...], preferred_element_type=f32)
    res_ref[...] = jnp.dot(h, rw_ref[...], preferred_element_type=f32)
    fab_ref[0] = feat[:, :16]
    fab_ref[1] = feat[:, 16:32]
    fc_ref[...] = jnp.concatenate(
        [feat[:, 32:], jnp.zeros((feat.shape[0], 8), f32)], axis=1)
    el_ref[...] = jnp.dot(feat, alt_ref[...], preferred_element_type=f32)
    er_ref[...] = jnp.dot(feat, art_ref[...], preferred_element_type=f32)


def _tc_layer2(acc1, den1, h0, b1, W2, resW2, al2, ar2):
    return _tc_call(
        _tc2_body,
        (jax.ShapeDtypeStruct((NC, NP, 16), f32),
         jax.ShapeDtypeStruct((NP, 16), f32),
         jax.ShapeDtypeStruct((NP, 1), f32),
         jax.ShapeDtypeStruct((NP, 1), f32),
         jax.ShapeDtypeStruct((NP, NCLS), f32),
         jax.ShapeDtypeStruct((NP, 1), f32)),
        [_rows((NC, _BR, 16)), _rows_last((NC, _BR)), _rows((_BR, HID)),
         _full((1, HID)), _full((HID, NCLS)), _full((HID, NCLS)),
         _full((NCLS, 1)), _full((NCLS, 1))],
        (_rows((NC, _BR, 16)), _rows((_BR, 16)), _rows((_BR, 1)),
         _rows((_BR, 1)), _rows((_BR, NCLS)), _rows((_BR, 1))),
        acc1, den1, h0, b1.reshape(1, HID), W2, resW2, al2.T, ar2.T)


def _tc3_body(acc_ref, accc_ref, den_ref, res_ref, b_ref, out_ref):
    den = den_ref[0] + den_ref[1] + EPS
    cc = accc_ref[0] + accc_ref[1]
    rst = jnp.concatenate([acc_ref[0], acc_ref[1], cc[:, :8]], axis=1)
    out_ref[...] = rst / den[:, None] + res_ref[...] + b_ref[...]


def _tc_final(acc2, accc2, den2, res2, b2):
    return _tc_call(
        _tc3_body,
        jax.ShapeDtypeStruct((NP, NCLS), f32),
        [_rows((NC, _BR, 16)), _rows((NC, _BR, 16)), _rows_last((NC, _BR)),
         _rows((_BR, NCLS)), _full((1, NCLS))],
        _rows((_BR, NCLS)),
        acc2, accc2, den2, res2, b2.reshape(1, NCLS))


def kernel(x, edge_index, W0, al0, ar0, b0, W1, al1, ar1, b1, W2, al2, ar2,
           b2, resW2):
    src = edge_index[0]
    dst = edge_index[1]
    xp = jnp.pad(x, ((0, NP - N), (0, 0)))

    fab0, el0, er0 = _tc_layer0(xp, W0, al0, ar0)
    s0 = _sc_gather_el(el0.reshape(NP), src)
    ee0, den0 = _sc_score(er0.reshape(NP), s0, dst)
    acc0 = _sc_agg(fab0, src, dst, ee0)

    fab1, el1, er1, h0 = _tc_layer1(acc0, den0, b0, W1, al1, ar1)
    s1 = _sc_gather_el(el1.reshape(NP), src)
    ee1, den1 = _sc_score(er1.reshape(NP), s1, dst)
    acc1 = _sc_agg(fab1, src, dst, ee1)

    fab2, fc2, el2, er2, res2, prior = _tc_layer2(
        acc1, den1, h0, b1, W2, resW2, al2, ar2)
    s2 = _sc_gather_el(el2.reshape(NP), src)
    ee2, den2 = _sc_score(er2.reshape(NP), s2, dst)
    acc2 = _sc_agg(fab2, src, dst, ee2)
    accc2 = _sc_agg_split(fc2, src, dst, ee2)
    alpha = _sc_alpha(den2, ee2, dst)

    logits = _tc_final(acc2, accc2, den2, res2, b2)
    return logits[:N], prior[:N, 0], alpha[:, None]


# speedup vs baseline: 1.0347x; 1.0347x over previous
import functools

import jax
import jax.numpy as jnp
from jax import lax
from jax.experimental import pallas as pl
from jax.experimental.pallas import tpu as pltpu
from jax.experimental.pallas import tpu_sc as plsc

N = 100000
E = 1600000
IN_DIM = 128
HID = 32
NCLS = 40
NEG = 0.2
EPS = 1e-9

NP = 100352
NT = 16
NC = 2
NW = NC * NT
STRIPE = NP // NT
ZR = 392
CH = 2000
CHA = 400
EPW = E // NW
EPT = E // NT

_mesh = plsc.VectorSubcoreMesh(core_axis_name="c", subcore_axis_name="s")
_sc_params = pltpu.CompilerParams(
    needs_layout_passes=False, use_tc_tiling_on_sc=False)
f32 = jnp.float32
i32 = jnp.int32


@functools.partial(
    pl.kernel,
    out_type=jax.ShapeDtypeStruct((E,), f32),
    mesh=_mesh,
    compiler_params=_sc_params,
    scratch_types=[
        pltpu.VMEM((NP,), f32),
        pltpu.VMEM((CH,), i32),
        pltpu.VMEM((CH,), f32),
    ],
)
def _sc_gather_el(el_hbm, src_hbm, s_hbm, tab, idxb, outb):
    cid = lax.axis_index("c")
    sid = lax.axis_index("s")
    wid = cid * NT + sid
    pltpu.sync_copy(el_hbm, tab)
    base0 = wid * EPW

    def chunk(ci, _):
        b = base0 + ci * CH
        pltpu.sync_copy(src_hbm.at[pl.ds(b, CH)], idxb)

        def grp(g, _):
            o = g * 16
            idx = idxb[pl.ds(o, 16)]
            outb[pl.ds(o, 16)] = plsc.load_gather(tab, [idx])
            return 0

        lax.fori_loop(0, CH // 16, grp, 0, unroll=4)
        pltpu.sync_copy(outb, s_hbm.at[pl.ds(b, CH)])
        return 0

    lax.fori_loop(0, EPW // CH, chunk, 0)


@functools.partial(
    pl.kernel,
    out_type=(
        jax.ShapeDtypeStruct((E,), f32),
        jax.ShapeDtypeStruct((NC, NP), f32),
    ),
    mesh=_mesh,
    compiler_params=_sc_params,
    scratch_types=[
        pltpu.VMEM((NP,), f32),
        pltpu.VMEM((CH,), i32),
        pltpu.VMEM((CH,), f32),
        pltpu.VMEM((CH,), f32),
        pltpu.VMEM((STRIPE,), f32),
        pltpu.VMEM_SHARED((NP,), f32),
    ],
)
def _sc_score(er_hbm, s_hbm, dst_hbm, ee_hbm, den_hbm,
              tab, dstb, sb, eeb, stage, den_sp):
    cid = lax.axis_index("c")
    sid = lax.axis_index("s")
    wid = cid * NT + sid
    pltpu.sync_copy(er_hbm, tab)

    def zgrp(g, _):
        stage[pl.ds(g * 16, 16)] = jnp.zeros((16,), f32)
        return 0

    lax.fori_loop(0, STRIPE // 16, zgrp, 0, unroll=4)
    pltpu.sync_copy(stage, den_sp.at[pl.ds(sid * STRIPE, STRIPE)])
    plsc.subcore_barrier()

    base0 = wid * EPW

    def chunk(ci, _):
        b = base0 + ci * CH
        pltpu.sync_copy(dst_hbm.at[pl.ds(b, CH)], dstb)
        pltpu.sync_copy(s_hbm.at[pl.ds(b, CH)], sb)

        def grp(g, _):
            o = g * 16
            idx = dstb[pl.ds(o, 16)]
            e = sb[pl.ds(o, 16)] + plsc.load_gather(tab, [idx])
            e = jnp.where(e > 0, e, NEG * e)
            eeb[pl.ds(o, 16)] = jnp.exp(e)
            return 0

        lax.fori_loop(0, CH // 16, grp, 0, unroll=4)
        pltpu.sync_copy(eeb, ee_hbm.at[pl.ds(b, CH)])
        pltpu.sync_copy(eeb, den_sp.at[dstb], add=True)
        return 0

    lax.fori_loop(0, EPW // CH, chunk, 0)

    plsc.subcore_barrier()
    off = sid * STRIPE
    pltpu.sync_copy(den_sp.at[pl.ds(off, STRIPE)], stage)
    pltpu.sync_copy(stage, den_hbm.at[cid, pl.ds(off, STRIPE)])


_AGG_SCRATCH = [
    pltpu.VMEM((2, CHA), i32),
    pltpu.VMEM((2, CHA), i32),
    pltpu.VMEM((2, CHA), f32),
    pltpu.VMEM((2, CHA, 16), f32),
    pltpu.VMEM((ZR, 16), f32),
    pltpu.VMEM_SHARED((NP, 16), f32),
    pltpu.SemaphoreType.DMA,
    pltpu.SemaphoreType.DMA,
]


def _agg_pipeline(tab, src_hbm, dst_hbm, w_hbm, base0, nch, ch,
                  idxs, idxd, wb, rows, acc_sp, gsems):
    def sync_idx(ci, p):
        b = base0 + ci * ch
        pltpu.sync_copy(src_hbm.at[pl.ds(b, ch)], idxs.at[p])
        pltpu.sync_copy(dst_hbm.at[pl.ds(b, ch)], idxd.at[p])
        pltpu.sync_copy(w_hbm.at[pl.ds(b, ch)], wb.at[p])

    def issue_gather(p):
        pltpu.async_copy(tab.at[idxs.at[p]], rows.at[p], gsems[p])

    def process(p):
        pltpu.make_async_copy(tab.at[idxs.at[p]], rows.at[p],
                              gsems[p]).wait()

        def grp(g, _):
            o = g * 16
            av = wb[p, pl.ds(o, 16)]
            for j in range(16):
                rows[p, o + j, :] = rows[p, o + j, :] * av[j]
            return 0

        lax.fori_loop(0, ch // 16, grp, 0)
        pltpu.sync_copy(rows.at[p], acc_sp.at[idxd.at[p]], add=True)

    sync_idx(0, 0)
    issue_gather(0)

    def pair(k, _):
        ci0 = 2 * k

        @pl.when(ci0 + 1 < nch)
        def _():
            sync_idx(ci0 + 1, 1)
            issue_gather(1)

        process(0)

        @pl.when(ci0 + 1 < nch)
        def _():
            @pl.when(ci0 + 2 < nch)
            def _():
                sync_idx(ci0 + 2, 0)
                issue_gather(0)

            process(1)

        return 0

    lax.fori_loop(0, (nch + 1) // 2, pair, 0)


def _acc_zero(stage, acc_sp, sid):
    def zgrp(g, _):
        stage[g, :] = jnp.zeros((16,), f32)
        return 0

    lax.fori_loop(0, ZR, zgrp, 0, unroll=4)
    for k in range(NT):
        pltpu.sync_copy(stage, acc_sp.at[pl.ds(sid * STRIPE + k * ZR, ZR)])
    plsc.subcore_barrier()


def _acc_writeback(stage, acc_sp, acc_hbm, cid, sid):
    plsc.subcore_barrier()
    for k in range(NT):
        off = sid * STRIPE + k * ZR
        pltpu.sync_copy(acc_sp.at[pl.ds(off, ZR)], stage)
        pltpu.sync_copy(stage, acc_hbm.at[cid, pl.ds(off, ZR)])


@functools.partial(
    pl.kernel,
    out_type=jax.ShapeDtypeStruct((NC, NP, 16), f32),
    mesh=_mesh,
    compiler_params=_sc_params,
    scratch_types=_AGG_SCRATCH,
)
def _sc_agg(fab_hbm, src_hbm, dst_hbm, w_hbm, acc_hbm,
            idxs, idxd, wb, rows, stage, acc_sp, gsem0, gsem1):
    cid = lax.axis_index("c")
    sid = lax.axis_index("s")
    _acc_zero(stage, acc_sp, sid)
    _agg_pipeline(fab_hbm.at[cid], src_hbm, dst_hbm, w_hbm, sid * EPT,
                  EPT // CHA, CHA, idxs, idxd, wb, rows, acc_sp,
                  (gsem0, gsem1))
    _acc_writeback(stage, acc_sp, acc_hbm, cid, sid)


@functools.partial(
    pl.kernel,
    out_type=jax.ShapeDtypeStruct((NC, NP, 16), f32),
    mesh=_mesh,
    compiler_params=_sc_params,
    scratch_types=_AGG_SCRATCH,
)
def _sc_agg_split(tab_hbm, src_hbm, dst_hbm, w_hbm, acc_hbm,
                  idxs, idxd, wb, rows, stage, acc_sp, gsem0, gsem1):
    cid = lax.axis_index("c")
    sid = lax.axis_index("s")
    _acc_zero(stage, acc_sp, sid)
    _agg_pipeline(tab_hbm, src_hbm, dst_hbm, w_hbm,
                  (cid * NT + sid) * EPW, EPW // CHA, CHA,
                  idxs, idxd, wb, rows, acc_sp, (gsem0, gsem1))
    _acc_writeback(stage, acc_sp, acc_hbm, cid, sid)


@functools.partial(
    pl.kernel,
    out_type=jax.ShapeDtypeStruct((E,), f32),
    mesh=_mesh,
    compiler_params=_sc_params,
    scratch_types=[
        pltpu.VMEM((NP,), f32),
        pltpu.VMEM((2048,), f32),
        pltpu.VMEM((2048,), f32),
        pltpu.VMEM((CH,), i32),
        pltpu.VMEM((CH,), f32),
        pltpu.VMEM((CH,), f32),
    ],
)
def _sc_alpha(den_hbm, ee_hbm, dst_hbm, alpha_hbm, tab, da, db, dstb, eeb, ab):
    cid = lax.axis_index("c")
    sid = lax.axis_index("s")
    wid = cid * NT + sid

    def tchunk(ci, _):
        o = ci * 2048
        pltpu.sync_copy(den_hbm.at[0, pl.ds(o, 2048)], da)
        pltpu.sync_copy(den_hbm.at[1, pl.ds(o, 2048)], db)

        def tgrp(g, _):
            go = g * 16
            tab[pl.ds(o + go, 16)] = (da[pl.ds(go, 16)] + db[pl.ds(go, 16)]
                                      + EPS)
            return 0

        lax.fori_loop(0, 2048 // 16, tgrp, 0, unroll=4)
        return 0

    lax.fori_loop(0, NP // 2048, tchunk, 0)

    base0 = wid * EPW

    def chunk(ci, _):
        b = base0 + ci * CH
        pltpu.sync_copy(dst_hbm.at[pl.ds(b, CH)], dstb)
        pltpu.sync_copy(ee_hbm.at[pl.ds(b, CH)], eeb)

        def grp(g, _):
            o = g * 16
            idx = dstb[pl.ds(o, 16)]
            ab[pl.ds(o, 16)] = eeb[pl.ds(o, 16)] / plsc.load_gather(tab, [idx])
            return 0

        lax.fori_loop(0, CH // 16, grp, 0, unroll=4)
        pltpu.sync_copy(ab, alpha_hbm.at[pl.ds(b, CH)])
        return 0

    lax.fori_loop(0, EPW // CH, chunk, 0)


_BR = 512
_GRID = NP // _BR


def _tc_call(body, out_shapes, in_specs, out_specs, *args):
    return pl.pallas_call(
        body,
        grid=(_GRID,),
        in_specs=in_specs,
        out_specs=out_specs,
        out_shape=out_shapes,
    )(*args)


def _full(shape):
    return pl.BlockSpec(shape, lambda i: tuple(0 for _ in shape))


def _rows(shape):
    nd = len(shape)
    return pl.BlockSpec(shape, lambda i, nd=nd: tuple(
        i if d == nd - 2 else 0 for d in range(nd)))


def _rows_last(shape):
    nd = len(shape)
    return pl.BlockSpec(shape, lambda i, nd=nd: tuple(
        i if d == nd - 1 else 0 for d in range(nd)))


def _tc0_body(x_ref, w_ref, alt_ref, art_ref, fab_ref, el_ref, er_ref):
    feat = jnp.dot(x_ref[...], w_ref[...], preferred_element_type=f32)
    fab_ref[0] = feat[:, :16]
    fab_ref[1] = feat[:, 16:]
    el_ref[...] = jnp.dot(feat, alt_ref[...], preferred_element_type=f32)
    er_ref[...] = jnp.dot(feat, art_ref[...], preferred_element_type=f32)


def _tc_layer0(xp, W0, al0, ar0):
    return _tc_call(
        _tc0_body,
        (jax.ShapeDtypeStruct((NC, NP, 16), f32),
         jax.ShapeDtypeStruct((NP, 1), f32),
         jax.ShapeDtypeStruct((NP, 1), f32)),
        [_rows((_BR, IN_DIM)), _full((IN_DIM, HID)), _full((HID, 1)),
         _full((HID, 1))],
        (_rows((NC, _BR, 16)), _rows((_BR, 1)), _rows((_BR, 1))),
        xp, W0, al0.T, ar0.T)


def _tc1_body(acc_ref, den_ref, b_ref, w_ref, alt_ref, art_ref,
              res_ref, fab_ref, el_ref, er_ref, h_ref):
    den = den_ref[0] + den_ref[1] + EPS
    h = jnp.concatenate([acc_ref[0], acc_ref[1]], axis=1) / den[:, None]
    h = h + b_ref[...]
    if res_ref is not None:
        h = h + res_ref[...]
    h_ref[...] = h
    feat = jnp.dot(h, w_ref[...], preferred_element_type=f32)
    fab_ref[0] = feat[:, :16]
    fab_ref[1] = feat[:, 16:]
    el_ref[...] = jnp.dot(feat, alt_ref[...], preferred_element_type=f32)
    er_ref[...] = jnp.dot(feat, art_ref[...], preferred_element_type=f32)


def _tc_layer1(acc0, den0, b0, W1, al1, ar1):
    body = functools.partial(_tc1_body, res_ref=None)

    def b2(acc_ref, den_ref, b_ref, w_ref, alt_ref, art_ref,
           fab_ref, el_ref, er_ref, h_ref):
        _tc1_body(acc_ref, den_ref, b_ref, w_ref, alt_ref, art_ref,
                  None, fab_ref, el_ref, er_ref, h_ref)

    return _tc_call(
        b2,
        (jax.ShapeDtypeStruct((NC, NP, 16), f32),
         jax.ShapeDtypeStruct((NP, 1), f32),
         jax.ShapeDtypeStruct((NP, 1), f32),
         jax.ShapeDtypeStruct((NP, HID), f32)),
        [_rows((NC, _BR, 16)), _rows_last((NC, _BR)), _full((1, HID)),
         _full((HID, HID)), _full((HID, 1)), _full((HID, 1))],
        (_rows((NC, _BR, 16)), _rows((_BR, 1)), _rows((_BR, 1)),
         _rows((_BR, HID))),
        acc0, den0, b0.reshape(1, HID), W1, al1.T, ar1.T)


def _tc2_body(acc_ref, den_ref, h0_ref, b_ref, w_ref, rw_ref, alt_ref,
              art_ref, fab_ref, fc_ref, el_ref, er_ref, res_ref, pri_ref):
    den = den_ref[0] + den_ref[1] + EPS
    h = jnp.concatenate([acc_ref[0], acc_ref[1]], axis=1) / den[:, None]
    h = h + h0_ref[...] + b_ref[...]
    pri_ref[...] = jnp.mean(h, axis=1, keepdims=True)
    feat = jnp.dot(h, w_ref[...], preferred_element_type=f32)
    res_ref[...] = jnp.dot(h, rw_ref[...], preferred_element_type=f32)
    fab_ref[0] = feat[:, :16]
    fab_ref[1] = feat[:, 16:32]
    fc_ref[...] = jnp.concatenate(
        [feat[:, 32:], jnp.zeros((feat.shape[0], 8), f32)], axis=1)
    el_ref[...] = jnp.dot(feat, alt_ref[...], preferred_element_type=f32)
    er_ref[...] = jnp.dot(feat, art_ref[...], preferred_element_type=f32)


def _tc_layer2(acc1, den1, h0, b1, W2, resW2, al2, ar2):
    return _tc_call(
        _tc2_body,
        (jax.ShapeDtypeStruct((NC, NP, 16), f32),
         jax.ShapeDtypeStruct((NP, 16), f32),
         jax.ShapeDtypeStruct((NP, 1), f32),
         jax.ShapeDtypeStruct((NP, 1), f32),
         jax.ShapeDtypeStruct((NP, NCLS), f32),
         jax.ShapeDtypeStruct((NP, 1), f32)),
        [_rows((NC, _BR, 16)), _rows_last((NC, _BR)), _rows((_BR, HID)),
         _full((1, HID)), _full((HID, NCLS)), _full((HID, NCLS)),
         _full((NCLS, 1)), _full((NCLS, 1))],
        (_rows((NC, _BR, 16)), _rows((_BR, 16)), _rows((_BR, 1)),
         _rows((_BR, 1)), _rows((_BR, NCLS)), _rows((_BR, 1))),
        acc1, den1, h0, b1.reshape(1, HID), W2, resW2, al2.T, ar2.T)


def _tc3_body(acc_ref, accc_ref, den_ref, res_ref, b_ref, out_ref):
    den = den_ref[0] + den_ref[1] + EPS
    cc = accc_ref[0] + accc_ref[1]
    rst = jnp.concatenate([acc_ref[0], acc_ref[1], cc[:, :8]], axis=1)
    out_ref[...] = rst / den[:, None] + res_ref[...] + b_ref[...]


def _tc_final(acc2, accc2, den2, res2, b2):
    return _tc_call(
        _tc3_body,
        jax.ShapeDtypeStruct((NP, NCLS), f32),
        [_rows((NC, _BR, 16)), _rows((NC, _BR, 16)), _rows_last((NC, _BR)),
         _rows((_BR, NCLS)), _full((1, NCLS))],
        _rows((_BR, NCLS)),
        acc2, accc2, den2, res2, b2.reshape(1, NCLS))


def kernel(x, edge_index, W0, al0, ar0, b0, W1, al1, ar1, b1, W2, al2, ar2,
           b2, resW2):
    src = edge_index[0]
    dst = edge_index[1]
    xp = jnp.pad(x, ((0, NP - N), (0, 0)))

    fab0, el0, er0 = _tc_layer0(xp, W0, al0, ar0)
    s0 = _sc_gather_el(el0.reshape(NP), src)
    ee0, den0 = _sc_score(er0.reshape(NP), s0, dst)
    acc0 = _sc_agg(fab0, src, dst, ee0)

    fab1, el1, er1, h0 = _tc_layer1(acc0, den0, b0, W1, al1, ar1)
    s1 = _sc_gather_el(el1.reshape(NP), src)
    ee1, den1 = _sc_score(er1.reshape(NP), s1, dst)
    acc1 = _sc_agg(fab1, src, dst, ee1)

    fab2, fc2, el2, er2, res2, prior = _tc_layer2(
        acc1, den1, h0, b1, W2, resW2, al2, ar2)
    s2 = _sc_gather_el(el2.reshape(NP), src)
    ee2, den2 = _sc_score(er2.reshape(NP), s2, dst)
    acc2 = _sc_agg(fab2, src, dst, ee2)
    accc2 = _sc_agg_split(fc2, src, dst, ee2)
    alpha = _sc_alpha(den2, ee2, dst)

    logits = _tc_final(acc2, accc2, den2, res2, b2)
    return logits[:N], prior[:N, 0], alpha[:, None]


# speedup vs baseline: 1.0573x; 1.0218x over previous
import functools

import jax
import jax.numpy as jnp
from jax import lax
from jax.experimental import pallas as pl
from jax.experimental.pallas import tpu as pltpu
from jax.experimental.pallas import tpu_sc as plsc

N = 100000
E = 1600000
IN_DIM = 128
HID = 32
NCLS = 40
NEG = 0.2
EPS = 1e-9

NP = 100352
NT = 16
NC = 2
NW = NC * NT
STRIPE = NP // NT
ZR = 392
CH = 2000
CHA = 400
EPW = E // NW
EPT = E // NT

_mesh = plsc.VectorSubcoreMesh(core_axis_name="c", subcore_axis_name="s")
_sc_params = pltpu.CompilerParams(
    needs_layout_passes=False, use_tc_tiling_on_sc=False)
f32 = jnp.float32
i32 = jnp.int32


@functools.partial(
    pl.kernel,
    out_type=jax.ShapeDtypeStruct((E,), f32),
    mesh=_mesh,
    compiler_params=_sc_params,
    scratch_types=[
        pltpu.VMEM((NP,), f32),
        pltpu.VMEM((CH,), i32),
        pltpu.VMEM((CH,), f32),
    ],
)
def _sc_gather_el(el_hbm, src_hbm, s_hbm, tab, idxb, outb):
    cid = lax.axis_index("c")
    sid = lax.axis_index("s")
    wid = cid * NT + sid
    pltpu.sync_copy(el_hbm, tab)
    base0 = wid * EPW

    def chunk(ci, _):
        b = base0 + ci * CH
        pltpu.sync_copy(src_hbm.at[pl.ds(b, CH)], idxb)

        def grp(g, _):
            o = g * 16
            idx = idxb[pl.ds(o, 16)]
            outb[pl.ds(o, 16)] = plsc.load_gather(tab, [idx])
            return 0

        lax.fori_loop(0, CH // 16, grp, 0, unroll=4)
        pltpu.sync_copy(outb, s_hbm.at[pl.ds(b, CH)])
        return 0

    lax.fori_loop(0, EPW // CH, chunk, 0)


@functools.partial(
    pl.kernel,
    out_type=(
        jax.ShapeDtypeStruct((E,), f32),
        jax.ShapeDtypeStruct((NC, NP), f32),
    ),
    mesh=_mesh,
    compiler_params=_sc_params,
    scratch_types=[
        pltpu.VMEM((NP,), f32),
        pltpu.VMEM((CH,), i32),
        pltpu.VMEM((CH,), f32),
        pltpu.VMEM((CH,), f32),
        pltpu.VMEM((STRIPE,), f32),
        pltpu.VMEM_SHARED((NP,), f32),
    ],
)
def _sc_score(er_hbm, s_hbm, dst_hbm, ee_hbm, den_hbm,
              tab, dstb, sb, eeb, stage, den_sp):
    cid = lax.axis_index("c")
    sid = lax.axis_index("s")
    wid = cid * NT + sid
    pltpu.sync_copy(er_hbm, tab)

    def zgrp(g, _):
        stage[pl.ds(g * 16, 16)] = jnp.zeros((16,), f32)
        return 0

    lax.fori_loop(0, STRIPE // 16, zgrp, 0, unroll=4)
    pltpu.sync_copy(stage, den_sp.at[pl.ds(sid * STRIPE, STRIPE)])
    plsc.subcore_barrier()

    base0 = wid * EPW

    def chunk(ci, _):
        b = base0 + ci * CH
        pltpu.sync_copy(dst_hbm.at[pl.ds(b, CH)], dstb)
        pltpu.sync_copy(s_hbm.at[pl.ds(b, CH)], sb)

        def grp(g, _):
            o = g * 16
            idx = dstb[pl.ds(o, 16)]
            e = sb[pl.ds(o, 16)] + plsc.load_gather(tab, [idx])
            e = jnp.where(e > 0, e, NEG * e)
            eeb[pl.ds(o, 16)] = jnp.exp(e)
            return 0

        lax.fori_loop(0, CH // 16, grp, 0, unroll=4)
        pltpu.sync_copy(eeb, ee_hbm.at[pl.ds(b, CH)])
        pltpu.sync_copy(eeb, den_sp.at[dstb], add=True)
        return 0

    lax.fori_loop(0, EPW // CH, chunk, 0)

    plsc.subcore_barrier()
    off = sid * STRIPE
    pltpu.sync_copy(den_sp.at[pl.ds(off, STRIPE)], stage)
    pltpu.sync_copy(stage, den_hbm.at[cid, pl.ds(off, STRIPE)])


_AGG_SCRATCH = [
    pltpu.VMEM((2, CHA), i32),
    pltpu.VMEM((2, CHA), i32),
    pltpu.VMEM((2, CHA), f32),
    pltpu.VMEM((2, CHA, 16), f32),
    pltpu.VMEM((ZR, 16), f32),
    pltpu.VMEM_SHARED((NP, 16), f32),
    pltpu.SemaphoreType.DMA,
    pltpu.SemaphoreType.DMA,
]


def _agg_pipeline(tab, src_hbm, dst_hbm, w_hbm, base0, nch, ch,
                  idxs, idxd, wb, rows, acc_sp, gsems):
    def sync_idx(ci, p):
        b = base0 + ci * ch
        pltpu.sync_copy(src_hbm.at[pl.ds(b, ch)], idxs.at[p])
        pltpu.sync_copy(dst_hbm.at[pl.ds(b, ch)], idxd.at[p])
        pltpu.sync_copy(w_hbm.at[pl.ds(b, ch)], wb.at[p])

    def issue_gather(p):
        pltpu.async_copy(tab.at[idxs.at[p]], rows.at[p], gsems[p])

    def process(p):
        pltpu.make_async_copy(tab.at[idxs.at[p]], rows.at[p],
                              gsems[p]).wait()

        for g in range(ch // 16):
            o = g * 16
            av = wb[p, pl.ds(o, 16)]
            for j in range(16):
                rows[p, o + j, :] = rows[p, o + j, :] * av[j]

        pltpu.sync_copy(rows.at[p], acc_sp.at[idxd.at[p]], add=True)

    sync_idx(0, 0)
    issue_gather(0)

    def pair(k, _):
        ci0 = 2 * k

        @pl.when(ci0 + 1 < nch)
        def _():
            sync_idx(ci0 + 1, 1)
            issue_gather(1)

        process(0)

        @pl.when(ci0 + 1 < nch)
        def _():
            @pl.when(ci0 + 2 < nch)
            def _():
                sync_idx(ci0 + 2, 0)
                issue_gather(0)

            process(1)

        return 0

    lax.fori_loop(0, (nch + 1) // 2, pair, 0)


def _acc_zero(stage, acc_sp, sid):
    def zgrp(g, _):
        stage[g, :] = jnp.zeros((16,), f32)
        return 0

    lax.fori_loop(0, ZR, zgrp, 0, unroll=4)
    for k in range(NT):
        pltpu.sync_copy(stage, acc_sp.at[pl.ds(sid * STRIPE + k * ZR, ZR)])
    plsc.subcore_barrier()


def _acc_writeback(stage, acc_sp, acc_hbm, cid, sid):
    plsc.subcore_barrier()
    for k in range(NT):
        off = sid * STRIPE + k * ZR
        pltpu.sync_copy(acc_sp.at[pl.ds(off, ZR)], stage)
        pltpu.sync_copy(stage, acc_hbm.at[cid, pl.ds(off, ZR)])


@functools.partial(
    pl.kernel,
    out_type=jax.ShapeDtypeStruct((NC, NP, 16), f32),
    mesh=_mesh,
    compiler_params=_sc_params,
    scratch_types=_AGG_SCRATCH,
)
def _sc_agg(fab_hbm, src_hbm, dst_hbm, w_hbm, acc_hbm,
            idxs, idxd, wb, rows, stage, acc_sp, gsem0, gsem1):
    cid = lax.axis_index("c")
    sid = lax.axis_index("s")
    _acc_zero(stage, acc_sp, sid)
    _agg_pipeline(fab_hbm.at[cid], src_hbm, dst_hbm, w_hbm, sid * EPT,
                  EPT // CHA, CHA, idxs, idxd, wb, rows, acc_sp,
                  (gsem0, gsem1))
    _acc_writeback(stage, acc_sp, acc_hbm, cid, sid)


@functools.partial(
    pl.kernel,
    out_type=jax.ShapeDtypeStruct((NC, NP, 16), f32),
    mesh=_mesh,
    compiler_params=_sc_params,
    scratch_types=_AGG_SCRATCH,
)
def _sc_agg_split(tab_hbm, src_hbm, dst_hbm, w_hbm, acc_hbm,
                  idxs, idxd, wb, rows, stage, acc_sp, gsem0, gsem1):
    cid = lax.axis_index("c")
    sid = lax.axis_index("s")
    _acc_zero(stage, acc_sp, sid)
    _agg_pipeline(tab_hbm, src_hbm, dst_hbm, w_hbm,
                  (cid * NT + sid) * EPW, EPW // CHA, CHA,
                  idxs, idxd, wb, rows, acc_sp, (gsem0, gsem1))
    _acc_writeback(stage, acc_sp, acc_hbm, cid, sid)


@functools.partial(
    pl.kernel,
    out_type=jax.ShapeDtypeStruct((E,), f32),
    mesh=_mesh,
    compiler_params=_sc_params,
    scratch_types=[
        pltpu.VMEM((NP,), f32),
        pltpu.VMEM((2048,), f32),
        pltpu.VMEM((2048,), f32),
        pltpu.VMEM((CH,), i32),
        pltpu.VMEM((CH,), f32),
        pltpu.VMEM((CH,), f32),
    ],
)
def _sc_alpha(den_hbm, ee_hbm, dst_hbm, alpha_hbm, tab, da, db, dstb, eeb, ab):
    cid = lax.axis_index("c")
    sid = lax.axis_index("s")
    wid = cid * NT + sid

    def tchunk(ci, _):
        o = ci * 2048
        pltpu.sync_copy(den_hbm.at[0, pl.ds(o, 2048)], da)
        pltpu.sync_copy(den_hbm.at[1, pl.ds(o, 2048)], db)

        def tgrp(g, _):
            go = g * 16
            tab[pl.ds(o + go, 16)] = (da[pl.ds(go, 16)] + db[pl.ds(go, 16)]
                                      + EPS)
            return 0

        lax.fori_loop(0, 2048 // 16, tgrp, 0, unroll=4)
        return 0

    lax.fori_loop(0, NP // 2048, tchunk, 0)

    base0 = wid * EPW

    def chunk(ci, _):
        b = base0 + ci * CH
        pltpu.sync_copy(dst_hbm.at[pl.ds(b, CH)], dstb)
        pltpu.sync_copy(ee_hbm.at[pl.ds(b, CH)], eeb)

        def grp(g, _):
            o = g * 16
            idx = dstb[pl.ds(o, 16)]
            ab[pl.ds(o, 16)] = eeb[pl.ds(o, 16)] / plsc.load_gather(tab, [idx])
            return 0

        lax.fori_loop(0, CH // 16, grp, 0, unroll=4)
        pltpu.sync_copy(ab, alpha_hbm.at[pl.ds(b, CH)])
        return 0

    lax.fori_loop(0, EPW // CH, chunk, 0)


_BR = 512
_GRID = NP // _BR


def _tc_call(body, out_shapes, in_specs, out_specs, *args):
    return pl.pallas_call(
        body,
        grid=(_GRID,),
        in_specs=in_specs,
        out_specs=out_specs,
        out_shape=out_shapes,
    )(*args)


def _full(shape):
    return pl.BlockSpec(shape, lambda i: tuple(0 for _ in shape))


def _rows(shape):
    nd = len(shape)
    return pl.BlockSpec(shape, lambda i, nd=nd: tuple(
        i if d == nd - 2 else 0 for d in range(nd)))


def _rows_last(shape):
    nd = len(shape)
    return pl.BlockSpec(shape, lambda i, nd=nd: tuple(
        i if d == nd - 1 else 0 for d in range(nd)))


def _tc0_body(x_ref, w_ref, alt_ref, art_ref, fab_ref, el_ref, er_ref):
    feat = jnp.dot(x_ref[...], w_ref[...], preferred_element_type=f32)
    fab_ref[0] = feat[:, :16]
    fab_ref[1] = feat[:, 16:]
    el_ref[...] = jnp.dot(feat, alt_ref[...], preferred_element_type=f32)
    er_ref[...] = jnp.dot(feat, art_ref[...], preferred_element_type=f32)


def _tc_layer0(xp, W0, al0, ar0):
    return _tc_call(
        _tc0_body,
        (jax.ShapeDtypeStruct((NC, NP, 16), f32),
         jax.ShapeDtypeStruct((NP, 1), f32),
         jax.ShapeDtypeStruct((NP, 1), f32)),
        [_rows((_BR, IN_DIM)), _full((IN_DIM, HID)), _full((HID, 1)),
         _full((HID, 1))],
        (_rows((NC, _BR, 16)), _rows((_BR, 1)), _rows((_BR, 1))),
        xp, W0, al0.T, ar0.T)


def _tc1_body(acc_ref, den_ref, b_ref, w_ref, alt_ref, art_ref,
              res_ref, fab_ref, el_ref, er_ref, h_ref):
    den = den_ref[0] + den_ref[1] + EPS
    h = jnp.concatenate([acc_ref[0], acc_ref[1]], axis=1) / den[:, None]
    h = h + b_ref[...]
    if res_ref is not None:
        h = h + res_ref[...]
    h_ref[...] = h
    feat = jnp.dot(h, w_ref[...], preferred_element_type=f32)
    fab_ref[0] = feat[:, :16]
    fab_ref[1] = feat[:, 16:]
    el_ref[...] = jnp.dot(feat, alt_ref[...], preferred_element_type=f32)
    er_ref[...] = jnp.dot(feat, art_ref[...], preferred_element_type=f32)


def _tc_layer1(acc0, den0, b0, W1, al1, ar1):
    body = functools.partial(_tc1_body, res_ref=None)

    def b2(acc_ref, den_ref, b_ref, w_ref, alt_ref, art_ref,
           fab_ref, el_ref, er_ref, h_ref):
        _tc1_body(acc_ref, den_ref, b_ref, w_ref, alt_ref, art_ref,
                  None, fab_ref, el_ref, er_ref, h_ref)

    return _tc_call(
        b2,
        (jax.ShapeDtypeStruct((NC, NP, 16), f32),
         jax.ShapeDtypeStruct((NP, 1), f32),
         jax.ShapeDtypeStruct((NP, 1), f32),
         jax.ShapeDtypeStruct((NP, HID), f32)),
        [_rows((NC, _BR, 16)), _rows_last((NC, _BR)), _full((1, HID)),
         _full((HID, HID)), _full((HID, 1)), _full((HID, 1))],
        (_rows((NC, _BR, 16)), _rows((_BR, 1)), _rows((_BR, 1)),
         _rows((_BR, HID))),
        acc0, den0, b0.reshape(1, HID), W1, al1.T, ar1.T)


def _tc2_body(acc_ref, den_ref, h0_ref, b_ref, w_ref, rw_ref, alt_ref,
              art_ref, fab_ref, fc_ref, el_ref, er_ref, res_ref, pri_ref):
    den = den_ref[0] + den_ref[1] + EPS
    h = jnp.concatenate([acc_ref[0], acc_ref[1]], axis=1) / den[:, None]
    h = h + h0_ref[...] + b_ref[...]
    pri_ref[...] = jnp.mean(h, axis=1, keepdims=True)
    feat = jnp.dot(h, w_ref[...], preferred_element_type=f32)
    res_ref[...] = jnp.dot(h, rw_ref[...], preferred_element_type=f32)
    fab_ref[0] = feat[:, :16]
    fab_ref[1] = feat[:, 16:32]
    fc_ref[...] = jnp.concatenate(
        [feat[:, 32:], jnp.zeros((feat.shape[0], 8), f32)], axis=1)
    el_ref[...] = jnp.dot(feat, alt_ref[...], preferred_element_type=f32)
    er_ref[...] = jnp.dot(feat, art_ref[...], preferred_element_type=f32)


def _tc_layer2(acc1, den1, h0, b1, W2, resW2, al2, ar2):
    return _tc_call(
        _tc2_body,
        (jax.ShapeDtypeStruct((NC, NP, 16), f32),
         jax.ShapeDtypeStruct((NP, 16), f32),
         jax.ShapeDtypeStruct((NP, 1), f32),
         jax.ShapeDtypeStruct((NP, 1), f32),
         jax.ShapeDtypeStruct((NP, NCLS), f32),
         jax.ShapeDtypeStruct((NP, 1), f32)),
        [_rows((NC, _BR, 16)), _rows_last((NC, _BR)), _rows((_BR, HID)),
         _full((1, HID)), _full((HID, NCLS)), _full((HID, NCLS)),
         _full((NCLS, 1)), _full((NCLS, 1))],
        (_rows((NC, _BR, 16)), _rows((_BR, 16)), _rows((_BR, 1)),
         _rows((_BR, 1)), _rows((_BR, NCLS)), _rows((_BR, 1))),
        acc1, den1, h0, b1.reshape(1, HID), W2, resW2, al2.T, ar2.T)


def _tc3_body(acc_ref, accc_ref, den_ref, res_ref, b_ref, out_ref):
    den = den_ref[0] + den_ref[1] + EPS
    cc = accc_ref[0] + accc_ref[1]
    rst = jnp.concatenate([acc_ref[0], acc_ref[1], cc[:, :8]], axis=1)
    out_ref[...] = rst / den[:, None] + res_ref[...] + b_ref[...]


def _tc_final(acc2, accc2, den2, res2, b2):
    return _tc_call(
        _tc3_body,
        jax.ShapeDtypeStruct((NP, NCLS), f32),
        [_rows((NC, _BR, 16)), _rows((NC, _BR, 16)), _rows_last((NC, _BR)),
         _rows((_BR, NCLS)), _full((1, NCLS))],
        _rows((_BR, NCLS)),
        acc2, accc2, den2, res2, b2.reshape(1, NCLS))


def kernel(x, edge_index, W0, al0, ar0, b0, W1, al1, ar1, b1, W2, al2, ar2,
           b2, resW2):
    src = edge_index[0]
    dst = edge_index[1]
    xp = jnp.pad(x, ((0, NP - N), (0, 0)))

    fab0, el0, er0 = _tc_layer0(xp, W0, al0, ar0)
    s0 = _sc_gather_el(el0.reshape(NP), src)
    ee0, den0 = _sc_score(er0.reshape(NP), s0, dst)
    acc0 = _sc_agg(fab0, src, dst, ee0)

    fab1, el1, er1, h0 = _tc_layer1(acc0, den0, b0, W1, al1, ar1)
    s1 = _sc_gather_el(el1.reshape(NP), src)
    ee1, den1 = _sc_score(er1.reshape(NP), s1, dst)
    acc1 = _sc_agg(fab1, src, dst, ee1)

    fab2, fc2, el2, er2, res2, prior = _tc_layer2(
        acc1, den1, h0, b1, W2, resW2, al2, ar2)
    s2 = _sc_gather_el(el2.reshape(NP), src)
    ee2, den2 = _sc_score(er2.reshape(NP), s2, dst)
    acc2 = _sc_agg(fab2, src, dst, ee2)
    accc2 = _sc_agg_split(fc2, src, dst, ee2)
    alpha = _sc_alpha(den2, ee2, dst)

    logits = _tc_final(acc2, accc2, den2, res2, b2)
    return logits[:N], prior[:N, 0], alpha[:, None]
